# Initial kernel scaffold; baseline (speedup 1.0000x reference)
import functools

import jax
import jax.numpy as jnp
from jax import lax
from jax.experimental import pallas as pl
from jax.experimental.pallas import tpu as pltpu
from jax.experimental.pallas import tpu_sc as plsc

N_NODES = 10000
D = 128
N_PAD = 10240
ROWS_PER_TILE = N_PAD // 16
E = 320000
NB_PER_TILE = 79
E_PAD = 128 * NB_PER_TILE * 32

_mesh = plsc.VectorSubcoreMesh(core_axis_name="c", subcore_axis_name="s")


@functools.partial(
    pl.kernel,
    out_type=jax.ShapeDtypeStruct((2 * N_PAD, 16), jnp.float32),
    mesh=_mesh,
    scratch_types=[
        pltpu.VMEM((NB_PER_TILE, 128), jnp.int32),
        pltpu.VMEM((128, 16), jnp.float32),
        pltpu.VMEM_SHARED((N_PAD, 16), jnp.float32),
    ],
)
def _deg_kernel(dst_hbm, out_hbm, dst_v, buf_v, acc_sh):
    c = lax.axis_index("c")
    s = lax.axis_index("s")
    w = c * 16 + s

    def _fill(val):
        def body(i, carry):
            buf_v[i, :] = jnp.full((16,), val, jnp.float32)
            return carry
        lax.fori_loop(0, 128, body, 0)

    _fill(0.0)
    r0 = s * ROWS_PER_TILE
    for k in range(ROWS_PER_TILE // 128):
        pltpu.sync_copy(buf_v, acc_sh.at[pl.ds(r0 + k * 128, 128)])
    plsc.subcore_barrier()

    _fill(1.0)
    pltpu.sync_copy(dst_hbm.at[pl.ds(w * NB_PER_TILE, NB_PER_TILE)], dst_v)

    def body(j, carry):
        pltpu.sync_copy(buf_v, acc_sh.at[dst_v.at[j]], add=True)
        return carry
    lax.fori_loop(0, NB_PER_TILE, body, 0)
    plsc.subcore_barrier()

    for k in range(ROWS_PER_TILE // 128):
        pltpu.sync_copy(acc_sh.at[pl.ds(r0 + k * 128, 128)], buf_v)
        pltpu.sync_copy(buf_v, out_hbm.at[pl.ds(c * N_PAD + r0 + k * 128, 128)])


@functools.partial(
    pl.kernel,
    out_type=jax.ShapeDtypeStruct((2 * N_PAD, D), jnp.float32),
    mesh=_mesh,
    scratch_types=[
        pltpu.VMEM((NB_PER_TILE, 128), jnp.int32),
        pltpu.VMEM((NB_PER_TILE, 128), jnp.int32),
        pltpu.VMEM((128, D), jnp.float32),
        pltpu.VMEM_SHARED((N_PAD, D), jnp.float32),
        pltpu.SemaphoreType.DMA,
    ],
)
def _agg_kernel(src_hbm, dst_hbm, h_hbm, out_hbm, src_v, dst_v, rows_v, acc_sh, sem):
    c = lax.axis_index("c")
    s = lax.axis_index("s")
    w = c * 16 + s

    def zbody(i, carry):
        for j in range(D // 16):
            rows_v[i, pl.ds(j * 16, 16)] = jnp.zeros((16,), jnp.float32)
        return carry
    lax.fori_loop(0, 128, zbody, 0)
    r0 = s * ROWS_PER_TILE
    for k in range(ROWS_PER_TILE // 128):
        pltpu.sync_copy(rows_v, acc_sh.at[pl.ds(r0 + k * 128, 128)])
    plsc.subcore_barrier()

    pltpu.sync_copy(src_hbm.at[pl.ds(w * NB_PER_TILE, NB_PER_TILE)], src_v)
    pltpu.sync_copy(dst_hbm.at[pl.ds(w * NB_PER_TILE, NB_PER_TILE)], dst_v)

    def body(j, carry):
        pltpu.async_copy(h_hbm.at[src_v.at[j]], rows_v, sem).wait()
        pltpu.sync_copy(rows_v, acc_sh.at[dst_v.at[j]], add=True)
        return carry
    lax.fori_loop(0, NB_PER_TILE, body, 0)
    plsc.subcore_barrier()

    for k in range(ROWS_PER_TILE // 128):
        pltpu.sync_copy(acc_sh.at[pl.ds(r0 + k * 128, 128)], rows_v)
        pltpu.sync_copy(rows_v, out_hbm.at[pl.ds(c * N_PAD + r0 + k * 128, 128)])


_R = 1024
_G = N_PAD // _R


def _tc1_body(x_ref, w_ref, degp_ref, h_ref, dis_ref):
    deg = jnp.sum(degp_ref[...], axis=(0, 2)) * (1.0 / 16.0) + 1.0
    dis = lax.rsqrt(deg)
    dis_ref[...] = dis
    h_ref[...] = jnp.dot(
        x_ref[...], w_ref[...], preferred_element_type=jnp.float32
    ) * dis[:, None]


_tc1 = pl.pallas_call(
    _tc1_body,
    grid=(_G,),
    in_specs=[
        pl.BlockSpec((_R, D), lambda i: (i, 0)),
        pl.BlockSpec((D, D), lambda i: (0, 0)),
        pl.BlockSpec((2, _R, 16), lambda i: (0, i, 0)),
    ],
    out_specs=[
        pl.BlockSpec((_R, D), lambda i: (i, 0)),
        pl.BlockSpec((_R,), lambda i: (i,)),
    ],
    out_shape=[
        jax.ShapeDtypeStruct((N_PAD, D), jnp.float32),
        jax.ShapeDtypeStruct((N_PAD,), jnp.float32),
    ],
)


def _tc2_body(acc_ref, h_ref, dis_ref, b_ref, w_ref, out_ref):
    dis = dis_ref[...]
    z = (acc_ref[0] + acc_ref[1] + h_ref[...]) * dis[:, None] + b_ref[...][None, :]
    z = jnp.maximum(z, 0.0)
    out_ref[...] = jnp.dot(
        z, w_ref[...], preferred_element_type=jnp.float32
    ) * dis[:, None]


_tc2 = pl.pallas_call(
    _tc2_body,
    grid=(_G,),
    in_specs=[
        pl.BlockSpec((2, _R, D), lambda i: (0, i, 0)),
        pl.BlockSpec((_R, D), lambda i: (i, 0)),
        pl.BlockSpec((_R,), lambda i: (i,)),
        pl.BlockSpec((D,), lambda i: (0,)),
        pl.BlockSpec((D, D), lambda i: (0, 0)),
    ],
    out_specs=pl.BlockSpec((_R, D), lambda i: (i, 0)),
    out_shape=jax.ShapeDtypeStruct((N_PAD, D), jnp.float32),
)


def _tc3_body(acc_ref, h_ref, dis_ref, b_ref, out_ref):
    dis = dis_ref[...]
    out_ref[...] = (
        (acc_ref[0] + acc_ref[1] + h_ref[...]) * dis[:, None] + b_ref[...][None, :]
    )


_tc3 = pl.pallas_call(
    _tc3_body,
    grid=(_G,),
    in_specs=[
        pl.BlockSpec((2, _R, D), lambda i: (0, i, 0)),
        pl.BlockSpec((_R, D), lambda i: (i, 0)),
        pl.BlockSpec((_R,), lambda i: (i,)),
        pl.BlockSpec((D,), lambda i: (0,)),
    ],
    out_specs=pl.BlockSpec((_R, D), lambda i: (i, 0)),
    out_shape=jax.ShapeDtypeStruct((N_PAD, D), jnp.float32),
)


def kernel(x, edge_index, W1, b1, W2, b2):
    src = edge_index[0].astype(jnp.int32)
    dst = edge_index[1].astype(jnp.int32)
    pad = jnp.full((E_PAD - E,), N_NODES, jnp.int32)
    src_p = jnp.concatenate([src, pad]).reshape(E_PAD // 128, 128)
    dst_p = jnp.concatenate([dst, pad]).reshape(E_PAD // 128, 128)
    x_p = jnp.pad(x, ((0, N_PAD - N_NODES), (0, 0)))

    degp = _deg_kernel(dst_p).reshape(2, N_PAD, 16)
    h1, dis = _tc1(x_p, W1, degp)
    acc1 = _agg_kernel(src_p, dst_p, h1).reshape(2, N_PAD, D)
    h2 = _tc2(acc1, h1, dis, b1, W2)
    acc2 = _agg_kernel(src_p, dst_p, h2).reshape(2, N_PAD, D)
    out = _tc3(acc2, h2, dis, b2)
    return out[:N_NODES]


# speedup vs baseline: 10.4699x; 10.4699x over previous
import functools

import jax
import jax.numpy as jnp
from jax import lax
from jax.experimental import pallas as pl
from jax.experimental.pallas import tpu as pltpu
from jax.experimental.pallas import tpu_sc as plsc

N_NODES = 10000
D = 128
DH = D // 2
N_PAD = 10240
ROWS_PER_TILE = N_PAD // 16
E = 320000
NB_DEG = 80
NB_AGG = 160
E_PAD = 128 * NB_DEG * 32

_mesh = plsc.VectorSubcoreMesh(core_axis_name="c", subcore_axis_name="s")


@functools.partial(
    pl.kernel,
    out_type=jax.ShapeDtypeStruct((32, N_PAD), jnp.float32),
    mesh=_mesh,
    compiler_params=pltpu.CompilerParams(needs_layout_passes=False),
    scratch_types=[
        pltpu.VMEM((NB_DEG, 128), jnp.int32),
        pltpu.VMEM((N_PAD,), jnp.float32),
    ],
)
def _deg_kernel(dst_hbm, out_hbm, dst_v, deg_v):
    c = lax.axis_index("c")
    s = lax.axis_index("s")
    w = c * 16 + s

    def zb(i, carry):
        deg_v[pl.ds(i * 16, 16)] = jnp.zeros((16,), jnp.float32)
        return carry
    lax.fori_loop(0, N_PAD // 16, zb, 0)

    pltpu.sync_copy(dst_hbm.at[pl.ds(w * NB_DEG, NB_DEG)], dst_v)
    ones = jnp.ones((16,), jnp.float32)

    def body(j, carry):
        for k in range(8):
            plsc.addupdate_scatter(deg_v, [dst_v[j, pl.ds(k * 16, 16)]], ones)
        return carry
    lax.fori_loop(0, NB_DEG, body, 0)

    pltpu.sync_copy(deg_v, out_hbm.at[w])


_NBUF = 2
_NGRP = NB_AGG // _NBUF


@functools.partial(
    pl.kernel,
    out_type=[jax.ShapeDtypeStruct((N_PAD, DH), jnp.float32),
              jax.ShapeDtypeStruct((N_PAD, DH), jnp.float32)],
    mesh=_mesh,
    compiler_params=pltpu.CompilerParams(use_tc_tiling_on_sc=False),
    scratch_types=[
        pltpu.VMEM((NB_AGG, 128), jnp.int32),
        pltpu.VMEM((NB_AGG, 128), jnp.int32),
        [pltpu.VMEM((128, DH), jnp.float32)] * _NBUF,
        pltpu.VMEM_SHARED((N_PAD, DH), jnp.float32),
        [pltpu.SemaphoreType.DMA] * _NBUF,
        [pltpu.SemaphoreType.DMA] * _NBUF,
    ],
)
def _agg_kernel(src2_hbm, dst_hbm, h_hbm, out0_hbm, out1_hbm, src_v, dst_v,
                rows_v, acc_sh, gsem, ssem):
    c = lax.axis_index("c")
    s = lax.axis_index("s")

    def zbody(i, carry):
        for j in range(DH // 16):
            rows_v[0][i, pl.ds(j * 16, 16)] = jnp.zeros((16,), jnp.float32)
        return carry
    lax.fori_loop(0, 128, zbody, 0)
    r0 = s * ROWS_PER_TILE
    for k in range(ROWS_PER_TILE // 128):
        pltpu.sync_copy(rows_v[0], acc_sh.at[pl.ds(r0 + k * 128, 128)])
    plsc.subcore_barrier()

    pltpu.sync_copy(src2_hbm.at[pl.ds((c * 16 + s) * NB_AGG, NB_AGG)], src_v)
    pltpu.sync_copy(dst_hbm.at[pl.ds(s * NB_AGG, NB_AGG)], dst_v)

    @pl.loop(0, _NGRP)
    def grp(g):
        base = g * _NBUF
        gds = [pltpu.async_copy(h_hbm.at[src_v.at[base + b]], rows_v[b],
                                gsem[b])
               for b in range(_NBUF)]
        sds = []
        for b in range(_NBUF):
            gds[b].wait()
            sds.append(pltpu.async_copy(rows_v[b], acc_sh.at[dst_v.at[base + b]],
                                        ssem[b], add=True))
        for d in sds:
            d.wait()

    plsc.subcore_barrier()

    for k in range(ROWS_PER_TILE // 128):
        pltpu.sync_copy(acc_sh.at[pl.ds(r0 + k * 128, 128)], rows_v[0])

        @pl.when(c == 0)
        def _():
            pltpu.sync_copy(rows_v[0], out0_hbm.at[pl.ds(r0 + k * 128, 128)])

        @pl.when(c == 1)
        def _():
            pltpu.sync_copy(rows_v[0], out1_hbm.at[pl.ds(r0 + k * 128, 128)])


_R = 1024
_G = N_PAD // _R


def _tc1_body(x_ref, w_ref, degp_ref, h_ref, dis_ref):
    deg = jnp.sum(degp_ref[...], axis=0) + 1.0
    dis = lax.rsqrt(deg)
    dis_ref[...] = dis
    h_ref[...] = jnp.dot(
        x_ref[...], w_ref[...], preferred_element_type=jnp.float32
    ) * dis[:, None]


_tc1 = pl.pallas_call(
    _tc1_body,
    grid=(_G,),
    in_specs=[
        pl.BlockSpec((_R, D), lambda i: (i, 0)),
        pl.BlockSpec((D, D), lambda i: (0, 0)),
        pl.BlockSpec((32, _R), lambda i: (0, i)),
    ],
    out_specs=[
        pl.BlockSpec((_R, D), lambda i: (i, 0)),
        pl.BlockSpec((_R,), lambda i: (i,)),
    ],
    out_shape=[
        jax.ShapeDtypeStruct((N_PAD, D), jnp.float32),
        jax.ShapeDtypeStruct((N_PAD,), jnp.float32),
    ],
)


def _tc2_body(acc_ref, h_ref, dis_ref, b_ref, w_ref, out_ref):
    dis = dis_ref[...]
    agg = jnp.concatenate([acc_ref[0], acc_ref[1]], axis=1) + h_ref[...]
    z = agg * dis[:, None] + b_ref[...][None, :]
    z = jnp.maximum(z, 0.0)
    out_ref[...] = jnp.dot(
        z, w_ref[...], preferred_element_type=jnp.float32
    ) * dis[:, None]


_tc2 = pl.pallas_call(
    _tc2_body,
    grid=(_G,),
    in_specs=[
        pl.BlockSpec((2, _R, DH), lambda i: (0, i, 0)),
        pl.BlockSpec((_R, D), lambda i: (i, 0)),
        pl.BlockSpec((_R,), lambda i: (i,)),
        pl.BlockSpec((D,), lambda i: (0,)),
        pl.BlockSpec((D, D), lambda i: (0, 0)),
    ],
    out_specs=pl.BlockSpec((_R, D), lambda i: (i, 0)),
    out_shape=jax.ShapeDtypeStruct((N_PAD, D), jnp.float32),
)


def _tc3_body(acc_ref, h_ref, dis_ref, b_ref, out_ref):
    dis = dis_ref[...]
    agg = jnp.concatenate([acc_ref[0], acc_ref[1]], axis=1) + h_ref[...]
    out_ref[...] = agg * dis[:, None] + b_ref[...][None, :]


_tc3 = pl.pallas_call(
    _tc3_body,
    grid=(_G,),
    in_specs=[
        pl.BlockSpec((2, _R, DH), lambda i: (0, i, 0)),
        pl.BlockSpec((_R, D), lambda i: (i, 0)),
        pl.BlockSpec((_R,), lambda i: (i,)),
        pl.BlockSpec((D,), lambda i: (0,)),
    ],
    out_specs=pl.BlockSpec((_R, D), lambda i: (i, 0)),
    out_shape=jax.ShapeDtypeStruct((N_PAD, D), jnp.float32),
)


def kernel(x, edge_index, W1, b1, W2, b2):
    src = edge_index[0].astype(jnp.int32)
    dst = edge_index[1].astype(jnp.int32)
    pad = jnp.full((E_PAD - E,), N_NODES, jnp.int32)
    src_p = jnp.concatenate([src, pad]).reshape(E_PAD // 128, 128)
    dst_p = jnp.concatenate([dst, pad]).reshape(E_PAD // 128, 128)
    x_p = jnp.pad(x, ((0, N_PAD - N_NODES), (0, 0)))

    src2 = jnp.concatenate([2 * src_p, 2 * src_p + 1])

    degp = _deg_kernel(dst_p)
    h1, dis = _tc1(x_p, W1, degp)
    a10, a11 = _agg_kernel(src2, dst_p, h1.reshape(2 * N_PAD, DH))
    acc1 = jnp.stack([a10, a11])
    h2 = _tc2(acc1, h1, dis, b1, W2)
    a20, a21 = _agg_kernel(src2, dst_p, h2.reshape(2 * N_PAD, DH))
    acc2 = jnp.stack([a20, a21])
    out = _tc3(acc2, h2, dis, b2)
    return out[:N_NODES]


# speedup vs baseline: 10.7519x; 1.0269x over previous
import functools

import jax
import jax.numpy as jnp
from jax import lax
from jax.experimental import pallas as pl
from jax.experimental.pallas import tpu as pltpu
from jax.experimental.pallas import tpu_sc as plsc

N_NODES = 10000
D = 128
DH = D // 2
N_PAD = 10240
ROWS_PER_TILE = N_PAD // 16
E = 320000
NB_DEG = 80
NB_AGG = 160
E_PAD = 128 * NB_DEG * 32

_mesh = plsc.VectorSubcoreMesh(core_axis_name="c", subcore_axis_name="s")


@functools.partial(
    pl.kernel,
    out_type=jax.ShapeDtypeStruct((32, N_PAD), jnp.float32),
    mesh=_mesh,
    compiler_params=pltpu.CompilerParams(needs_layout_passes=False),
    scratch_types=[
        pltpu.VMEM((NB_DEG, 128), jnp.int32),
        pltpu.VMEM((N_PAD,), jnp.float32),
    ],
)
def _deg_kernel(dst_hbm, out_hbm, dst_v, deg_v):
    c = lax.axis_index("c")
    s = lax.axis_index("s")
    w = c * 16 + s

    def zb(i, carry):
        deg_v[pl.ds(i * 16, 16)] = jnp.zeros((16,), jnp.float32)
        return carry
    lax.fori_loop(0, N_PAD // 16, zb, 0)

    pltpu.sync_copy(dst_hbm.at[pl.ds(w * NB_DEG, NB_DEG)], dst_v)
    ones = jnp.ones((16,), jnp.float32)

    def body(j, carry):
        for k in range(8):
            plsc.addupdate_scatter(deg_v, [dst_v[j, pl.ds(k * 16, 16)]], ones)
        return carry
    lax.fori_loop(0, NB_DEG, body, 0)

    pltpu.sync_copy(deg_v, out_hbm.at[w])


_NBUF = 4
_NGRP = NB_AGG // _NBUF


@functools.partial(
    pl.kernel,
    out_type=[jax.ShapeDtypeStruct((N_PAD, DH), jnp.float32),
              jax.ShapeDtypeStruct((N_PAD, DH), jnp.float32)],
    mesh=_mesh,
    compiler_params=pltpu.CompilerParams(use_tc_tiling_on_sc=False),
    scratch_types=[
        pltpu.VMEM((NB_AGG, 128), jnp.int32),
        pltpu.VMEM((NB_AGG, 128), jnp.int32),
        [pltpu.VMEM((128, DH), jnp.float32)] * _NBUF,
        pltpu.VMEM_SHARED((N_PAD, DH), jnp.float32),
        [pltpu.SemaphoreType.DMA] * _NBUF,
        [pltpu.SemaphoreType.DMA] * _NBUF,
    ],
)
def _agg_kernel(src2_hbm, dst_hbm, h_hbm, out0_hbm, out1_hbm, src_v, dst_v,
                rows_v, acc_sh, gsem, ssem):
    c = lax.axis_index("c")
    s = lax.axis_index("s")

    def zbody(i, carry):
        for j in range(DH // 16):
            rows_v[0][i, pl.ds(j * 16, 16)] = jnp.zeros((16,), jnp.float32)
        return carry
    lax.fori_loop(0, 128, zbody, 0)
    r0 = s * ROWS_PER_TILE
    for k in range(ROWS_PER_TILE // 128):
        pltpu.sync_copy(rows_v[0], acc_sh.at[pl.ds(r0 + k * 128, 128)])
    plsc.subcore_barrier()

    pltpu.sync_copy(src2_hbm.at[pl.ds((c * 16 + s) * NB_AGG, NB_AGG)], src_v)
    pltpu.sync_copy(dst_hbm.at[pl.ds(s * NB_AGG, NB_AGG)], dst_v)

    @pl.loop(0, _NGRP)
    def grp(g):
        base = g * _NBUF
        gds = [pltpu.async_copy(h_hbm.at[src_v.at[base + b]], rows_v[b],
                                gsem[b])
               for b in range(_NBUF)]
        sds = []
        for b in range(_NBUF):
            gds[b].wait()
            sds.append(pltpu.async_copy(rows_v[b], acc_sh.at[dst_v.at[base + b]],
                                        ssem[b], add=True))
        for d in sds:
            d.wait()

    plsc.subcore_barrier()

    for k in range(ROWS_PER_TILE // 128):
        pltpu.sync_copy(acc_sh.at[pl.ds(r0 + k * 128, 128)], rows_v[0])

        @pl.when(c == 0)
        def _():
            pltpu.sync_copy(rows_v[0], out0_hbm.at[pl.ds(r0 + k * 128, 128)])

        @pl.when(c == 1)
        def _():
            pltpu.sync_copy(rows_v[0], out1_hbm.at[pl.ds(r0 + k * 128, 128)])


_R = 1024
_G = N_PAD // _R


def _tc1_body(x_ref, w_ref, degp_ref, h_ref, dis_ref):
    deg = jnp.sum(degp_ref[...], axis=0) + 1.0
    dis = lax.rsqrt(deg)
    dis_ref[...] = dis
    h_ref[...] = jnp.dot(
        x_ref[...], w_ref[...], preferred_element_type=jnp.float32
    ) * dis[:, None]


_tc1 = pl.pallas_call(
    _tc1_body,
    grid=(_G,),
    in_specs=[
        pl.BlockSpec((_R, D), lambda i: (i, 0)),
        pl.BlockSpec((D, D), lambda i: (0, 0)),
        pl.BlockSpec((32, _R), lambda i: (0, i)),
    ],
    out_specs=[
        pl.BlockSpec((_R, D), lambda i: (i, 0)),
        pl.BlockSpec((_R,), lambda i: (i,)),
    ],
    out_shape=[
        jax.ShapeDtypeStruct((N_PAD, D), jnp.float32),
        jax.ShapeDtypeStruct((N_PAD,), jnp.float32),
    ],
)


def _tc2_body(acc_ref, h_ref, dis_ref, b_ref, w_ref, out_ref):
    dis = dis_ref[...]
    agg = jnp.concatenate([acc_ref[0], acc_ref[1]], axis=1) + h_ref[...]
    z = agg * dis[:, None] + b_ref[...][None, :]
    z = jnp.maximum(z, 0.0)
    out_ref[...] = jnp.dot(
        z, w_ref[...], preferred_element_type=jnp.float32
    ) * dis[:, None]


_tc2 = pl.pallas_call(
    _tc2_body,
    grid=(_G,),
    in_specs=[
        pl.BlockSpec((2, _R, DH), lambda i: (0, i, 0)),
        pl.BlockSpec((_R, D), lambda i: (i, 0)),
        pl.BlockSpec((_R,), lambda i: (i,)),
        pl.BlockSpec((D,), lambda i: (0,)),
        pl.BlockSpec((D, D), lambda i: (0, 0)),
    ],
    out_specs=pl.BlockSpec((_R, D), lambda i: (i, 0)),
    out_shape=jax.ShapeDtypeStruct((N_PAD, D), jnp.float32),
)


def _tc3_body(acc_ref, h_ref, dis_ref, b_ref, out_ref):
    dis = dis_ref[...]
    agg = jnp.concatenate([acc_ref[0], acc_ref[1]], axis=1) + h_ref[...]
    out_ref[...] = agg * dis[:, None] + b_ref[...][None, :]


_tc3 = pl.pallas_call(
    _tc3_body,
    grid=(_G,),
    in_specs=[
        pl.BlockSpec((2, _R, DH), lambda i: (0, i, 0)),
        pl.BlockSpec((_R, D), lambda i: (i, 0)),
        pl.BlockSpec((_R,), lambda i: (i,)),
        pl.BlockSpec((D,), lambda i: (0,)),
    ],
    out_specs=pl.BlockSpec((_R, D), lambda i: (i, 0)),
    out_shape=jax.ShapeDtypeStruct((N_PAD, D), jnp.float32),
)


def kernel(x, edge_index, W1, b1, W2, b2):
    src = edge_index[0].astype(jnp.int32)
    dst = edge_index[1].astype(jnp.int32)
    pad = jnp.full((E_PAD - E,), N_NODES, jnp.int32)
    src_p = jnp.concatenate([src, pad]).reshape(E_PAD // 128, 128)
    dst_p = jnp.concatenate([dst, pad]).reshape(E_PAD // 128, 128)
    x_p = jnp.pad(x, ((0, N_PAD - N_NODES), (0, 0)))

    src2 = jnp.concatenate([2 * src_p, 2 * src_p + 1])

    degp = _deg_kernel(dst_p)
    h1, dis = _tc1(x_p, W1, degp)
    a10, a11 = _agg_kernel(src2, dst_p, h1.reshape(2 * N_PAD, DH))
    acc1 = jnp.stack([a10, a11])
    h2 = _tc2(acc1, h1, dis, b1, W2)
    a20, a21 = _agg_kernel(src2, dst_p, h2.reshape(2 * N_PAD, DH))
    acc2 = jnp.stack([a20, a21])
    out = _tc3(acc2, h2, dis, b2)
    return out[:N_NODES]


# speedup vs baseline: 11.3070x; 1.0516x over previous
import functools

import jax
import jax.numpy as jnp
from jax import lax
from jax.experimental import pallas as pl
from jax.experimental.pallas import tpu as pltpu
from jax.experimental.pallas import tpu_sc as plsc

N_NODES = 10000
D = 128
DH = D // 2
N_PAD = 10240
ROWS_PER_TILE = N_PAD // 16
E = 320000
NB_DEG = 80
NB_AGG = 160
E_PAD = 128 * NB_DEG * 32

_mesh = plsc.VectorSubcoreMesh(core_axis_name="c", subcore_axis_name="s")


@functools.partial(
    pl.kernel,
    out_type=jax.ShapeDtypeStruct((32, N_PAD), jnp.float32),
    mesh=_mesh,
    compiler_params=pltpu.CompilerParams(needs_layout_passes=False),
    scratch_types=[
        pltpu.VMEM((NB_DEG, 128), jnp.int32),
        pltpu.VMEM((N_PAD,), jnp.float32),
    ],
)
def _deg_kernel(dst_hbm, out_hbm, dst_v, deg_v):
    c = lax.axis_index("c")
    s = lax.axis_index("s")
    w = c * 16 + s

    def zb(i, carry):
        deg_v[pl.ds(i * 16, 16)] = jnp.zeros((16,), jnp.float32)
        return carry
    lax.fori_loop(0, N_PAD // 16, zb, 0)

    pltpu.sync_copy(dst_hbm.at[pl.ds(w * NB_DEG, NB_DEG)], dst_v)
    ones = jnp.ones((16,), jnp.float32)

    def body(j, carry):
        for k in range(8):
            plsc.addupdate_scatter(deg_v, [dst_v[j, pl.ds(k * 16, 16)]], ones)
        return carry
    lax.fori_loop(0, NB_DEG, body, 0)

    pltpu.sync_copy(deg_v, out_hbm.at[w])


_NBUF = 4
_NGRP = NB_AGG // _NBUF


@functools.partial(
    pl.kernel,
    out_type=[jax.ShapeDtypeStruct((N_PAD, DH), jnp.float32),
              jax.ShapeDtypeStruct((N_PAD, DH), jnp.float32)],
    mesh=_mesh,
    compiler_params=pltpu.CompilerParams(use_tc_tiling_on_sc=False),
    scratch_types=[
        pltpu.VMEM((NB_AGG, 128), jnp.int32),
        pltpu.VMEM((NB_AGG, 128), jnp.int32),
        [pltpu.VMEM((128, DH), jnp.float32)] * _NBUF,
        pltpu.VMEM_SHARED((N_PAD, DH), jnp.float32),
        [pltpu.SemaphoreType.DMA] * _NBUF,
        [pltpu.SemaphoreType.DMA] * _NBUF,
    ],
)
def _agg_kernel(src2_hbm, dst_hbm, h_hbm, out0_hbm, out1_hbm, src_v, dst_v,
                rows_v, acc_sh, gsem, ssem):
    c = lax.axis_index("c")
    s = lax.axis_index("s")

    def zbody(i, carry):
        for j in range(DH // 16):
            rows_v[0][i, pl.ds(j * 16, 16)] = jnp.zeros((16,), jnp.float32)
        return carry
    lax.fori_loop(0, 128, zbody, 0)
    r0 = s * ROWS_PER_TILE
    for k in range(ROWS_PER_TILE // 128):
        pltpu.sync_copy(rows_v[0], acc_sh.at[pl.ds(r0 + k * 128, 128)])
    plsc.subcore_barrier()

    pltpu.sync_copy(src2_hbm.at[pl.ds((c * 16 + s) * NB_AGG, NB_AGG)], src_v)
    pltpu.sync_copy(dst_hbm.at[pl.ds(s * NB_AGG, NB_AGG)], dst_v)

    def gwait(b):
        pltpu.make_async_copy(h_hbm.at[src_v.at[0]], rows_v[b], gsem[b]).wait()

    def swait(b):
        pltpu.make_async_copy(rows_v[b], acc_sh.at[dst_v.at[0]], ssem[b]).wait()

    for b in range(_NBUF):
        pltpu.async_copy(h_hbm.at[src_v.at[b]], rows_v[b], gsem[b])

    @pl.loop(0, _NGRP)
    def grp(g):
        base = g * _NBUF
        for b in range(_NBUF):
            gwait(b)
            pltpu.async_copy(rows_v[b], acc_sh.at[dst_v.at[base + b]],
                             ssem[b], add=True)
        for b in range(_NBUF):
            swait(b)

            @pl.when(g < _NGRP - 1)
            def _():
                pltpu.async_copy(h_hbm.at[src_v.at[base + _NBUF + b]],
                                 rows_v[b], gsem[b])

    plsc.subcore_barrier()

    for k in range(ROWS_PER_TILE // 128):
        pltpu.sync_copy(acc_sh.at[pl.ds(r0 + k * 128, 128)], rows_v[0])

        @pl.when(c == 0)
        def _():
            pltpu.sync_copy(rows_v[0], out0_hbm.at[pl.ds(r0 + k * 128, 128)])

        @pl.when(c == 1)
        def _():
            pltpu.sync_copy(rows_v[0], out1_hbm.at[pl.ds(r0 + k * 128, 128)])


_R = 1024
_G = N_PAD // _R


def _tc1_body(x_ref, w_ref, degp_ref, h_ref, dis_ref):
    deg = jnp.sum(degp_ref[...], axis=0) + 1.0
    dis = lax.rsqrt(deg)
    dis_ref[...] = dis
    h_ref[...] = jnp.dot(
        x_ref[...], w_ref[...], preferred_element_type=jnp.float32
    ) * dis[:, None]


_tc1 = pl.pallas_call(
    _tc1_body,
    grid=(_G,),
    in_specs=[
        pl.BlockSpec((_R, D), lambda i: (i, 0)),
        pl.BlockSpec((D, D), lambda i: (0, 0)),
        pl.BlockSpec((32, _R), lambda i: (0, i)),
    ],
    out_specs=[
        pl.BlockSpec((_R, D), lambda i: (i, 0)),
        pl.BlockSpec((_R,), lambda i: (i,)),
    ],
    out_shape=[
        jax.ShapeDtypeStruct((N_PAD, D), jnp.float32),
        jax.ShapeDtypeStruct((N_PAD,), jnp.float32),
    ],
)


def _tc2_body(acc_ref, h_ref, dis_ref, b_ref, w_ref, out_ref):
    dis = dis_ref[...]
    agg = jnp.concatenate([acc_ref[0], acc_ref[1]], axis=1) + h_ref[...]
    z = agg * dis[:, None] + b_ref[...][None, :]
    z = jnp.maximum(z, 0.0)
    out_ref[...] = jnp.dot(
        z, w_ref[...], preferred_element_type=jnp.float32
    ) * dis[:, None]


_tc2 = pl.pallas_call(
    _tc2_body,
    grid=(_G,),
    in_specs=[
        pl.BlockSpec((2, _R, DH), lambda i: (0, i, 0)),
        pl.BlockSpec((_R, D), lambda i: (i, 0)),
        pl.BlockSpec((_R,), lambda i: (i,)),
        pl.BlockSpec((D,), lambda i: (0,)),
        pl.BlockSpec((D, D), lambda i: (0, 0)),
    ],
    out_specs=pl.BlockSpec((_R, D), lambda i: (i, 0)),
    out_shape=jax.ShapeDtypeStruct((N_PAD, D), jnp.float32),
)


def _tc3_body(acc_ref, h_ref, dis_ref, b_ref, out_ref):
    dis = dis_ref[...]
    agg = jnp.concatenate([acc_ref[0], acc_ref[1]], axis=1) + h_ref[...]
    out_ref[...] = agg * dis[:, None] + b_ref[...][None, :]


_tc3 = pl.pallas_call(
    _tc3_body,
    grid=(_G,),
    in_specs=[
        pl.BlockSpec((2, _R, DH), lambda i: (0, i, 0)),
        pl.BlockSpec((_R, D), lambda i: (i, 0)),
        pl.BlockSpec((_R,), lambda i: (i,)),
        pl.BlockSpec((D,), lambda i: (0,)),
    ],
    out_specs=pl.BlockSpec((_R, D), lambda i: (i, 0)),
    out_shape=jax.ShapeDtypeStruct((N_PAD, D), jnp.float32),
)


def kernel(x, edge_index, W1, b1, W2, b2):
    src = edge_index[0].astype(jnp.int32)
    dst = edge_index[1].astype(jnp.int32)
    pad = jnp.full((E_PAD - E,), N_NODES, jnp.int32)
    src_p = jnp.concatenate([src, pad]).reshape(E_PAD // 128, 128)
    dst_p = jnp.concatenate([dst, pad]).reshape(E_PAD // 128, 128)
    x_p = jnp.pad(x, ((0, N_PAD - N_NODES), (0, 0)))

    src2 = jnp.concatenate([2 * src_p, 2 * src_p + 1])

    degp = _deg_kernel(dst_p)
    h1, dis = _tc1(x_p, W1, degp)
    a10, a11 = _agg_kernel(src2, dst_p, h1.reshape(2 * N_PAD, DH))
    acc1 = jnp.stack([a10, a11])
    h2 = _tc2(acc1, h1, dis, b1, W2)
    a20, a21 = _agg_kernel(src2, dst_p, h2.reshape(2 * N_PAD, DH))
    acc2 = jnp.stack([a20, a21])
    out = _tc3(acc2, h2, dis, b2)
    return out[:N_NODES]


# speedup vs baseline: 11.3732x; 1.0059x over previous
import functools

import jax
import jax.numpy as jnp
from jax import lax
from jax.experimental import pallas as pl
from jax.experimental.pallas import tpu as pltpu
from jax.experimental.pallas import tpu_sc as plsc

N_NODES = 10000
D = 128
DH = D // 2
N_PAD = 10240
ROWS_PER_TILE = N_PAD // 16
E = 320000
NB_DEG = 80
NB_AGG = 160
E_PAD = 128 * NB_DEG * 32

_mesh = plsc.VectorSubcoreMesh(core_axis_name="c", subcore_axis_name="s")


@functools.partial(
    pl.kernel,
    out_type=jax.ShapeDtypeStruct((32, N_PAD), jnp.float32),
    mesh=_mesh,
    compiler_params=pltpu.CompilerParams(needs_layout_passes=False),
    scratch_types=[
        pltpu.VMEM((NB_DEG, 128), jnp.int32),
        pltpu.VMEM((N_PAD,), jnp.float32),
    ],
)
def _deg_kernel(dst_hbm, out_hbm, dst_v, deg_v):
    c = lax.axis_index("c")
    s = lax.axis_index("s")
    w = c * 16 + s

    def zb(i, carry):
        deg_v[pl.ds(i * 16, 16)] = jnp.zeros((16,), jnp.float32)
        return carry
    lax.fori_loop(0, N_PAD // 16, zb, 0)

    pltpu.sync_copy(dst_hbm.at[pl.ds(w * NB_DEG, NB_DEG)], dst_v)
    ones = jnp.ones((16,), jnp.float32)

    def body(j, carry):
        for k in range(8):
            plsc.addupdate_scatter(deg_v, [dst_v[j, pl.ds(k * 16, 16)]], ones)
        return carry
    lax.fori_loop(0, NB_DEG, body, 0)

    pltpu.sync_copy(deg_v, out_hbm.at[w])


_NBUF = 5
_NGRP = NB_AGG // _NBUF


@functools.partial(
    pl.kernel,
    out_type=[jax.ShapeDtypeStruct((N_PAD, DH), jnp.float32),
              jax.ShapeDtypeStruct((N_PAD, DH), jnp.float32)],
    mesh=_mesh,
    compiler_params=pltpu.CompilerParams(use_tc_tiling_on_sc=False),
    scratch_types=[
        pltpu.VMEM((NB_AGG, 128), jnp.int32),
        pltpu.VMEM((NB_AGG, 128), jnp.int32),
        [pltpu.VMEM((128, DH), jnp.float32)] * _NBUF,
        pltpu.VMEM_SHARED((N_PAD, DH), jnp.float32),
        [pltpu.SemaphoreType.DMA] * _NBUF,
        [pltpu.SemaphoreType.DMA] * _NBUF,
    ],
)
def _agg_kernel(src2_hbm, dst_hbm, h_hbm, out0_hbm, out1_hbm, src_v, dst_v,
                rows_v, acc_sh, gsem, ssem):
    c = lax.axis_index("c")
    s = lax.axis_index("s")

    def zbody(i, carry):
        for j in range(DH // 16):
            rows_v[0][i, pl.ds(j * 16, 16)] = jnp.zeros((16,), jnp.float32)
        return carry
    lax.fori_loop(0, 128, zbody, 0)
    r0 = s * ROWS_PER_TILE
    for k in range(ROWS_PER_TILE // 128):
        pltpu.sync_copy(rows_v[0], acc_sh.at[pl.ds(r0 + k * 128, 128)])
    plsc.subcore_barrier()

    pltpu.sync_copy(src2_hbm.at[pl.ds((c * 16 + s) * NB_AGG, NB_AGG)], src_v)
    pltpu.sync_copy(dst_hbm.at[pl.ds(s * NB_AGG, NB_AGG)], dst_v)

    def gwait(b):
        pltpu.make_async_copy(h_hbm.at[src_v.at[0]], rows_v[b], gsem[b]).wait()

    def swait(b):
        pltpu.make_async_copy(rows_v[b], acc_sh.at[dst_v.at[0]], ssem[b]).wait()

    for b in range(_NBUF):
        pltpu.async_copy(h_hbm.at[src_v.at[b]], rows_v[b], gsem[b])

    @pl.loop(0, _NGRP)
    def grp(g):
        base = g * _NBUF
        for b in range(_NBUF):
            gwait(b)
            pltpu.async_copy(rows_v[b], acc_sh.at[dst_v.at[base + b]],
                             ssem[b], add=True)
        for b in range(_NBUF):
            swait(b)

            @pl.when(g < _NGRP - 1)
            def _():
                pltpu.async_copy(h_hbm.at[src_v.at[base + _NBUF + b]],
                                 rows_v[b], gsem[b])

    plsc.subcore_barrier()

    for k in range(ROWS_PER_TILE // 128):
        pltpu.sync_copy(acc_sh.at[pl.ds(r0 + k * 128, 128)], rows_v[0])

        @pl.when(c == 0)
        def _():
            pltpu.sync_copy(rows_v[0], out0_hbm.at[pl.ds(r0 + k * 128, 128)])

        @pl.when(c == 1)
        def _():
            pltpu.sync_copy(rows_v[0], out1_hbm.at[pl.ds(r0 + k * 128, 128)])


_R = 1024
_G = N_PAD // _R


def _tc1_body(x_ref, w_ref, degp_ref, h_ref, dis_ref):
    deg = jnp.sum(degp_ref[...], axis=0) + 1.0
    dis = lax.rsqrt(deg)
    dis_ref[...] = dis
    h_ref[...] = jnp.dot(
        x_ref[...], w_ref[...], preferred_element_type=jnp.float32
    ) * dis[:, None]


_tc1 = pl.pallas_call(
    _tc1_body,
    grid=(_G,),
    in_specs=[
        pl.BlockSpec((_R, D), lambda i: (i, 0)),
        pl.BlockSpec((D, D), lambda i: (0, 0)),
        pl.BlockSpec((32, _R), lambda i: (0, i)),
    ],
    out_specs=[
        pl.BlockSpec((_R, D), lambda i: (i, 0)),
        pl.BlockSpec((_R,), lambda i: (i,)),
    ],
    out_shape=[
        jax.ShapeDtypeStruct((N_PAD, D), jnp.float32),
        jax.ShapeDtypeStruct((N_PAD,), jnp.float32),
    ],
)


def _tc2_body(acc_ref, h_ref, dis_ref, b_ref, w_ref, out_ref):
    dis = dis_ref[...]
    agg = jnp.concatenate([acc_ref[0], acc_ref[1]], axis=1) + h_ref[...]
    z = agg * dis[:, None] + b_ref[...][None, :]
    z = jnp.maximum(z, 0.0)
    out_ref[...] = jnp.dot(
        z, w_ref[...], preferred_element_type=jnp.float32
    ) * dis[:, None]


_tc2 = pl.pallas_call(
    _tc2_body,
    grid=(_G,),
    in_specs=[
        pl.BlockSpec((2, _R, DH), lambda i: (0, i, 0)),
        pl.BlockSpec((_R, D), lambda i: (i, 0)),
        pl.BlockSpec((_R,), lambda i: (i,)),
        pl.BlockSpec((D,), lambda i: (0,)),
        pl.BlockSpec((D, D), lambda i: (0, 0)),
    ],
    out_specs=pl.BlockSpec((_R, D), lambda i: (i, 0)),
    out_shape=jax.ShapeDtypeStruct((N_PAD, D), jnp.float32),
)


def _tc3_body(acc_ref, h_ref, dis_ref, b_ref, out_ref):
    dis = dis_ref[...]
    agg = jnp.concatenate([acc_ref[0], acc_ref[1]], axis=1) + h_ref[...]
    out_ref[...] = agg * dis[:, None] + b_ref[...][None, :]


_tc3 = pl.pallas_call(
    _tc3_body,
    grid=(_G,),
    in_specs=[
        pl.BlockSpec((2, _R, DH), lambda i: (0, i, 0)),
        pl.BlockSpec((_R, D), lambda i: (i, 0)),
        pl.BlockSpec((_R,), lambda i: (i,)),
        pl.BlockSpec((D,), lambda i: (0,)),
    ],
    out_specs=pl.BlockSpec((_R, D), lambda i: (i, 0)),
    out_shape=jax.ShapeDtypeStruct((N_PAD, D), jnp.float32),
)


def kernel(x, edge_index, W1, b1, W2, b2):
    src = edge_index[0].astype(jnp.int32)
    dst = edge_index[1].astype(jnp.int32)
    pad = jnp.full((E_PAD - E,), N_NODES, jnp.int32)
    src_p = jnp.concatenate([src, pad]).reshape(E_PAD // 128, 128)
    dst_p = jnp.concatenate([dst, pad]).reshape(E_PAD // 128, 128)
    x_p = jnp.pad(x, ((0, N_PAD - N_NODES), (0, 0)))

    src2 = jnp.concatenate([2 * src_p, 2 * src_p + 1])

    degp = _deg_kernel(dst_p)
    h1, dis = _tc1(x_p, W1, degp)
    a10, a11 = _agg_kernel(src2, dst_p, h1.reshape(2 * N_PAD, DH))
    acc1 = jnp.stack([a10, a11])
    h2 = _tc2(acc1, h1, dis, b1, W2)
    a20, a21 = _agg_kernel(src2, dst_p, h2.reshape(2 * N_PAD, DH))
    acc2 = jnp.stack([a20, a21])
    out = _tc3(acc2, h2, dis, b2)
    return out[:N_NODES]


# speedup vs baseline: 14.4813x; 1.2733x over previous
import functools

import jax
import jax.numpy as jnp
from jax import lax
from jax.experimental import pallas as pl
from jax.experimental.pallas import tpu as pltpu
from jax.experimental.pallas import tpu_sc as plsc

N_NODES = 10000
D = 128
DH = D // 2
N_PAD = 10240
ROWS_PER_TILE = N_PAD // 16
E = 320000
NB_DEG = 80
NB_AGG = 160
E_PAD = 128 * NB_DEG * 32

_mesh = plsc.VectorSubcoreMesh(core_axis_name="c", subcore_axis_name="s")


@functools.partial(
    pl.kernel,
    out_type=jax.ShapeDtypeStruct((32, N_PAD), jnp.float32),
    mesh=_mesh,
    compiler_params=pltpu.CompilerParams(needs_layout_passes=False),
    scratch_types=[
        pltpu.VMEM((NB_DEG, 128), jnp.int32),
        pltpu.VMEM((N_PAD,), jnp.float32),
    ],
)
def _deg_kernel(dst_hbm, out_hbm, dst_v, deg_v):
    c = lax.axis_index("c")
    s = lax.axis_index("s")
    w = c * 16 + s

    def zb(i, carry):
        deg_v[pl.ds(i * 16, 16)] = jnp.zeros((16,), jnp.float32)
        return carry
    lax.fori_loop(0, N_PAD // 16, zb, 0)

    pltpu.sync_copy(dst_hbm.at[pl.ds(w * NB_DEG, NB_DEG)], dst_v)
    ones = jnp.ones((16,), jnp.float32)

    def body(j, carry):
        for k in range(8):
            plsc.addupdate_scatter(deg_v, [dst_v[j, pl.ds(k * 16, 16)]], ones)
        return carry
    lax.fori_loop(0, NB_DEG, body, 0)

    pltpu.sync_copy(deg_v, out_hbm.at[w])


_NBUF = 5
_NGRP = NB_AGG // _NBUF


@functools.partial(
    pl.kernel,
    out_type=[jax.ShapeDtypeStruct((N_PAD, DH), jnp.float32),
              jax.ShapeDtypeStruct((N_PAD, DH), jnp.float32)],
    mesh=_mesh,
    compiler_params=pltpu.CompilerParams(use_tc_tiling_on_sc=False),
    scratch_types=[
        pltpu.VMEM((NB_AGG, 128), jnp.int32),
        pltpu.VMEM((NB_AGG, 128), jnp.int32),
        [pltpu.VMEM((128, DH), jnp.float32)] * _NBUF,
        pltpu.VMEM_SHARED((N_PAD, DH), jnp.float32),
        [pltpu.SemaphoreType.DMA] * _NBUF,
        [pltpu.SemaphoreType.DMA] * _NBUF,
    ],
)
def _agg_kernel(src_hbm, dst_hbm, h0_hbm, h1_hbm, out0_hbm, out1_hbm,
                src_v, dst_v, rows_v, acc_sh, gsem, ssem):
    c = lax.axis_index("c")
    s = lax.axis_index("s")

    def zbody(i, carry):
        for j in range(DH // 16):
            rows_v[0][i, pl.ds(j * 16, 16)] = jnp.zeros((16,), jnp.float32)
        return carry
    lax.fori_loop(0, 128, zbody, 0)
    r0 = s * ROWS_PER_TILE
    for k in range(ROWS_PER_TILE // 128):
        pltpu.sync_copy(rows_v[0], acc_sh.at[pl.ds(r0 + k * 128, 128)])
    plsc.subcore_barrier()

    pltpu.sync_copy(src_hbm.at[pl.ds(s * NB_AGG, NB_AGG)], src_v)
    pltpu.sync_copy(dst_hbm.at[pl.ds(s * NB_AGG, NB_AGG)], dst_v)

    def gwait(b, h_hbm):
        pltpu.make_async_copy(h_hbm.at[src_v.at[0]], rows_v[b], gsem[b]).wait()

    def swait(b):
        pltpu.make_async_copy(rows_v[b], acc_sh.at[dst_v.at[0]], ssem[b]).wait()

    def run(h_hbm):
        for b in range(_NBUF):
            pltpu.async_copy(h_hbm.at[src_v.at[b]], rows_v[b], gsem[b])

        @pl.loop(0, _NGRP)
        def grp(g):
            base = g * _NBUF
            for b in range(_NBUF):
                gwait(b, h_hbm)
                pltpu.async_copy(rows_v[b], acc_sh.at[dst_v.at[base + b]],
                                 ssem[b], add=True)
            for b in range(_NBUF):
                swait(b)

                @pl.when(g < _NGRP - 1)
                def _():
                    pltpu.async_copy(h_hbm.at[src_v.at[base + _NBUF + b]],
                                     rows_v[b], gsem[b])

    @pl.when(c == 0)
    def _():
        run(h0_hbm)

    @pl.when(c == 1)
    def _():
        run(h1_hbm)

    plsc.subcore_barrier()

    for k in range(ROWS_PER_TILE // 128):
        pltpu.sync_copy(acc_sh.at[pl.ds(r0 + k * 128, 128)], rows_v[0])

        @pl.when(c == 0)
        def _():
            pltpu.sync_copy(rows_v[0], out0_hbm.at[pl.ds(r0 + k * 128, 128)])

        @pl.when(c == 1)
        def _():
            pltpu.sync_copy(rows_v[0], out1_hbm.at[pl.ds(r0 + k * 128, 128)])


_R = 1024
_G = N_PAD // _R


def _tc1_body(x_ref, w_ref, degp_ref, h0_ref, h1_ref, dis_ref):
    deg = jnp.sum(degp_ref[...], axis=0) + 1.0
    dis = lax.rsqrt(deg)
    dis_ref[...] = dis
    hp = jnp.dot(
        x_ref[...], w_ref[...], preferred_element_type=jnp.float32
    ) * dis[:, None]
    h0_ref[...] = hp[:, :DH]
    h1_ref[...] = hp[:, DH:]


_tc1 = pl.pallas_call(
    _tc1_body,
    grid=(_G,),
    in_specs=[
        pl.BlockSpec((_R, D), lambda i: (i, 0)),
        pl.BlockSpec((D, D), lambda i: (0, 0)),
        pl.BlockSpec((32, _R), lambda i: (0, i)),
    ],
    out_specs=[
        pl.BlockSpec((_R, DH), lambda i: (i, 0)),
        pl.BlockSpec((_R, DH), lambda i: (i, 0)),
        pl.BlockSpec((_R,), lambda i: (i,)),
    ],
    out_shape=[
        jax.ShapeDtypeStruct((N_PAD, DH), jnp.float32),
        jax.ShapeDtypeStruct((N_PAD, DH), jnp.float32),
        jax.ShapeDtypeStruct((N_PAD,), jnp.float32),
    ],
)


def _tc2_body(a0_ref, a1_ref, h0_ref, h1_ref, dis_ref, b_ref, w_ref,
              o0_ref, o1_ref):
    dis = dis_ref[...]
    agg = jnp.concatenate([a0_ref[...] + h0_ref[...],
                           a1_ref[...] + h1_ref[...]], axis=1)
    z = agg * dis[:, None] + b_ref[...][None, :]
    z = jnp.maximum(z, 0.0)
    hp = jnp.dot(
        z, w_ref[...], preferred_element_type=jnp.float32
    ) * dis[:, None]
    o0_ref[...] = hp[:, :DH]
    o1_ref[...] = hp[:, DH:]


_tc2 = pl.pallas_call(
    _tc2_body,
    grid=(_G,),
    in_specs=[
        pl.BlockSpec((_R, DH), lambda i: (i, 0)),
        pl.BlockSpec((_R, DH), lambda i: (i, 0)),
        pl.BlockSpec((_R, DH), lambda i: (i, 0)),
        pl.BlockSpec((_R, DH), lambda i: (i, 0)),
        pl.BlockSpec((_R,), lambda i: (i,)),
        pl.BlockSpec((D,), lambda i: (0,)),
        pl.BlockSpec((D, D), lambda i: (0, 0)),
    ],
    out_specs=[
        pl.BlockSpec((_R, DH), lambda i: (i, 0)),
        pl.BlockSpec((_R, DH), lambda i: (i, 0)),
    ],
    out_shape=[
        jax.ShapeDtypeStruct((N_PAD, DH), jnp.float32),
        jax.ShapeDtypeStruct((N_PAD, DH), jnp.float32),
    ],
)


def _tc3_body(a0_ref, a1_ref, h0_ref, h1_ref, dis_ref, b_ref, out_ref):
    dis = dis_ref[...]
    agg = jnp.concatenate([a0_ref[...] + h0_ref[...],
                           a1_ref[...] + h1_ref[...]], axis=1)
    out_ref[...] = agg * dis[:, None] + b_ref[...][None, :]


_tc3 = pl.pallas_call(
    _tc3_body,
    grid=(_G,),
    in_specs=[
        pl.BlockSpec((_R, DH), lambda i: (i, 0)),
        pl.BlockSpec((_R, DH), lambda i: (i, 0)),
        pl.BlockSpec((_R, DH), lambda i: (i, 0)),
        pl.BlockSpec((_R, DH), lambda i: (i, 0)),
        pl.BlockSpec((_R,), lambda i: (i,)),
        pl.BlockSpec((D,), lambda i: (0,)),
    ],
    out_specs=pl.BlockSpec((_R, D), lambda i: (i, 0)),
    out_shape=jax.ShapeDtypeStruct((N_PAD, D), jnp.float32),
)


def kernel(x, edge_index, W1, b1, W2, b2):
    src = edge_index[0].astype(jnp.int32)
    dst = edge_index[1].astype(jnp.int32)
    pad = jnp.full((E_PAD - E,), N_NODES, jnp.int32)
    src_p = jnp.concatenate([src, pad]).reshape(E_PAD // 128, 128)
    dst_p = jnp.concatenate([dst, pad]).reshape(E_PAD // 128, 128)
    x_p = jnp.pad(x, ((0, N_PAD - N_NODES), (0, 0)))

    degp = _deg_kernel(dst_p)
    h10, h11, dis = _tc1(x_p, W1, degp)
    a10, a11 = _agg_kernel(src_p, dst_p, h10, h11)
    h20, h21 = _tc2(a10, a11, h10, h11, dis, b1, W2)
    a20, a21 = _agg_kernel(src_p, dst_p, h20, h21)
    out = _tc3(a20, a21, h20, h21, dis, b2)
    return out[:N_NODES]
